# Initial kernel scaffold; baseline (speedup 1.0000x reference)
import jax
import jax.numpy as jnp
from jax.experimental import pallas as pl


def kernel(position_ids, embed_table):
    raise NotImplementedError("write your pallas kernel here")


# speedup vs baseline: 3.6161x; 3.6161x over previous
import functools

import jax
import jax.numpy as jnp
from jax import lax
from jax.experimental import pallas as pl
from jax.experimental.pallas import tpu as pltpu
from jax.experimental.pallas import tpu_sc as plsc

NC = 2
NS = 16
NW = NC * NS
B = 4 * 8192
D = 128
BPW = B // NW
CHUNK = 128
NCHUNK = BPW // CHUNK


def _make_gather():
    mesh = plsc.VectorSubcoreMesh(core_axis_name="c", subcore_axis_name="s")

    @functools.partial(
        pl.kernel,
        mesh=mesh,
        out_type=jax.ShapeDtypeStruct((B, D), jnp.float32),
        scratch_types=[
            pltpu.VMEM((NCHUNK, CHUNK), jnp.int32),
            pltpu.VMEM((CHUNK, D), jnp.float32),
            pltpu.VMEM((CHUNK, D), jnp.float32),
            pltpu.SemaphoreType.DMA,
        ],
    )
    def gather_kernel(idx_hbm, table_hbm, out_hbm, idx_v, rows0, rows1, gsem):
        wid = lax.axis_index("s") * NC + lax.axis_index("c")
        base = wid * BPW
        pltpu.sync_copy(idx_hbm.at[wid], idx_v)
        bufs = (rows0, rows1)
        gathers = [None] * NCHUNK
        gathers[0] = pltpu.async_copy(table_hbm.at[idx_v.at[0]], bufs[0], gsem)
        for j in range(NCHUNK):
            gathers[j].wait()
            if j + 1 < NCHUNK:
                gathers[j + 1] = pltpu.async_copy(
                    table_hbm.at[idx_v.at[j + 1]], bufs[(j + 1) % 2], gsem)
            pltpu.sync_copy(bufs[j % 2], out_hbm.at[pl.ds(base + j * CHUNK, CHUNK)])

    return gather_kernel


_gather = _make_gather()


def kernel(position_ids, embed_table):
    idx = position_ids.astype(jnp.int32).reshape(NW, NCHUNK, CHUNK)
    out = _gather(idx, embed_table)
    return out.reshape(position_ids.shape + (D,))


# speedup vs baseline: 4.0060x; 1.1078x over previous
import functools

import jax
import jax.numpy as jnp
from jax import lax
from jax.experimental import pallas as pl
from jax.experimental.pallas import tpu as pltpu
from jax.experimental.pallas import tpu_sc as plsc

NC = 2
NS = 16
NW = NC * NS
B = 4 * 8192
D = 128
BPW = B // NW
CHUNK = 128
NCHUNK = BPW // CHUNK
NBUF = 4


def _make_gather():
    mesh = plsc.VectorSubcoreMesh(core_axis_name="c", subcore_axis_name="s")

    @functools.partial(
        pl.kernel,
        mesh=mesh,
        out_type=jax.ShapeDtypeStruct((B, D), jnp.float32),
        scratch_types=[
            pltpu.VMEM((NCHUNK, CHUNK), jnp.int32),
            pltpu.VMEM((NBUF, CHUNK, D), jnp.float32),
            pltpu.SemaphoreType.DMA,
            pltpu.SemaphoreType.DMA,
        ],
    )
    def gather_kernel(idx_hbm, table_hbm, out_hbm, idx_v, rows_v, gsem, wsem):
        wid = lax.axis_index("s") * NC + lax.axis_index("c")
        base = wid * BPW
        pltpu.sync_copy(idx_hbm.at[wid], idx_v)
        gathers = [None] * NCHUNK
        writes = [None] * NCHUNK
        for j in range(NBUF - 1):
            gathers[j] = pltpu.async_copy(
                table_hbm.at[idx_v.at[j]], rows_v.at[j % NBUF], gsem)
        for j in range(NCHUNK):
            gathers[j].wait()
            writes[j] = pltpu.async_copy(
                rows_v.at[j % NBUF], out_hbm.at[pl.ds(base + j * CHUNK, CHUNK)], wsem)
            nxt = j + NBUF - 1
            if nxt < NCHUNK:
                if nxt - NBUF >= 0:
                    writes[nxt - NBUF].wait()
                gathers[nxt] = pltpu.async_copy(
                    table_hbm.at[idx_v.at[nxt]], rows_v.at[nxt % NBUF], gsem)
        for j in range(NCHUNK - NBUF, NCHUNK):
            if j >= 0 and writes[j] is not None:
                writes[j].wait()

    return gather_kernel


_gather = _make_gather()


def kernel(position_ids, embed_table):
    idx = position_ids.astype(jnp.int32).reshape(NW, NCHUNK, CHUNK)
    out = _gather(idx, embed_table)
    return out.reshape(position_ids.shape + (D,))
